# baseline (speedup 1.0000x reference)
import functools

import jax
import jax.numpy as jnp
from jax import lax
from jax.experimental import pallas as pl
from jax.experimental.pallas import tpu as pltpu
from jax.experimental.pallas import tpu_sc as plsc

N = 50000
D = 256
B = 128
BLK = 5000
NBLK = N // BLK

NW = 16
NJ = 25
LW = 128
PW = NJ * LW
NP = NW * PW
SEG = 144


def _tc_body(pred_ref, tgt_ref, out_ref):
    diff = pred_ref[...] - tgt_ref[...]
    row_sum = jnp.sum(diff * diff, axis=1)
    out_ref[...] = row_sum.reshape(1, 1, BLK)


def _tc_row_mse(pred, target):
    return pl.pallas_call(
        _tc_body,
        grid=(NBLK,),
        in_specs=[
            pl.BlockSpec((BLK, D), lambda i: (i, 0)),
            pl.BlockSpec((BLK, D), lambda i: (i, 0)),
        ],
        out_specs=pl.BlockSpec((1, 1, BLK), lambda i: (i, 0, 0)),
        out_shape=jax.ShapeDtypeStruct((NBLK, 1, BLK), jnp.float32),
    )(pred, target)


@functools.partial(
    pl.kernel,
    out_type=jax.ShapeDtypeStruct((16,), jnp.float32),
    mesh=plsc.VectorSubcoreMesh(
        core_axis_name="c", subcore_axis_name="s", num_cores=1),
    scratch_types=[
        pltpu.VMEM((NJ, LW), jnp.float32),
        pltpu.VMEM((NJ, LW), jnp.int32),
        pltpu.VMEM((LW,), jnp.float32),
        pltpu.VMEM((SEG,), jnp.float32),
        pltpu.VMEM((SEG,), jnp.float32),
        pltpu.VMEM((16,), jnp.float32),
        pltpu.VMEM((16,), jnp.float32),
        pltpu.VMEM((16,), jnp.int32),
        pltpu.VMEM_SHARED((SEG,), jnp.float32),
        pltpu.VMEM_SHARED((SEG,), jnp.float32),
        pltpu.VMEM_SHARED((16,), jnp.float32),
    ],
)
def _sc_segment_mean(rs_hbm, idx_hbm, out_hbm,
                     vals_v, idx_v, ones_v, red_s, red_c, out_v,
                     tmp_v, zidx_v, sh_sums, sh_cnts, sh_res):
    w = lax.axis_index("s")
    pltpu.sync_copy(rs_hbm.at[w], vals_v)
    pltpu.sync_copy(idx_hbm.at[w], idx_v)

    for k in range(LW // 16):
        ones_v[pl.ds(k * 16, 16)] = jnp.ones((16,), jnp.float32)

    @pl.when(w == 0)
    def _init():
        for j in range(SEG // 16):
            red_s[pl.ds(j * 16, 16)] = jnp.zeros((16,), jnp.float32)
        pltpu.sync_copy(red_s, sh_sums)
        pltpu.sync_copy(red_s, sh_cnts)

    plsc.subcore_barrier()

    for j in range(NJ):
        pltpu.sync_copy(vals_v.at[j], sh_sums.at[idx_v.at[j]], add=True)
        pltpu.sync_copy(ones_v, sh_cnts.at[idx_v.at[j]], add=True)

    plsc.subcore_barrier()

    @pl.when(w == 0)
    def _finish():
        pltpu.sync_copy(sh_sums, red_s)
        pltpu.sync_copy(sh_cnts, red_c)
        tot = jnp.zeros((16,), jnp.float32)
        for j in range(B // 16):
            s_j = red_s[pl.ds(j * 16, 16)]
            c_j = red_c[pl.ds(j * 16, 16)]
            tot = tot + s_j / jnp.maximum(c_j, 1.0)
        tmp_v[...] = tot / jnp.float32(D * B)
        zidx_v[...] = jnp.zeros((16,), jnp.int32)
        out_v[...] = jnp.zeros((16,), jnp.float32)
        pltpu.sync_copy(out_v, sh_res)
        pltpu.sync_copy(tmp_v, sh_res.at[zidx_v], add=True)
        pltpu.sync_copy(sh_res, out_v)
        pltpu.sync_copy(out_v, out_hbm)


def kernel(pred, target, batch_idx, batch_size):
    del batch_size
    rs = _tc_row_mse(pred, target).reshape(N)
    rs_pad = jnp.concatenate(
        [rs, jnp.zeros((NP - N,), jnp.float32)]).reshape(NW, NJ, LW)
    idx_pad = jnp.concatenate([
        batch_idx.astype(jnp.int32),
        jnp.full((NP - N,), B, jnp.int32),
    ]).reshape(NW, NJ, LW)
    out = _sc_segment_mean(rs_pad, idx_pad)
    return out[0]
